# baseline (speedup 1.0000x reference)
import functools

import jax
import jax.numpy as jnp
from jax import lax
from jax.experimental import pallas as pl
from jax.experimental.pallas import tpu as pltpu
from jax.experimental.pallas import tpu_sc as plsc

B = 4096
NSF = 26
NDF = 13
V = 100000
NC = 2
NSUB = 16
NW = NC * NSUB
TB = B // NW
L = 16
NCH = TB // L

_mesh = plsc.VectorSubcoreMesh(
    core_axis_name="c", subcore_axis_name="s", num_cores=NC, num_subcores=NSUB
)


@functools.partial(
    pl.kernel,
    out_type=jax.ShapeDtypeStruct((B,), jnp.float32),
    mesh=_mesh,
    scratch_types=[
        pltpu.VMEM((NSF, TB), jnp.float32),
        pltpu.VMEM((NDF, TB), jnp.float32),
        pltpu.VMEM((NDF, L), jnp.float32),
        pltpu.VMEM((NSF, TB), jnp.int32),
        pltpu.VMEM((NSF, TB), jnp.float32),
        pltpu.VMEM((TB,), jnp.float32),
        pltpu.SemaphoreType.DMA,
    ],
)
def _linear_part(xs_hbm, xd_hbm, w_hbm, tbl_hbm, out_hbm,
                 xs_v, xd_v, w_v, idx_v, emb_v, acc_v, sem):
    wid = lax.axis_index("s") * NC + lax.axis_index("c")
    base = wid * TB

    pltpu.sync_copy(xs_hbm.at[wid], xs_v)

    for f in range(NSF):
        off = f * V
        for j in range(NCH):
            sl = pl.ds(j * L, L)
            idx_v[f, sl] = xs_v[f, sl].astype(jnp.int32) + off

    copies = [
        pltpu.async_copy(tbl_hbm.at[idx_v.at[f]], emb_v.at[f], sem)
        for f in range(NSF)
    ]

    pltpu.sync_copy(xd_hbm.at[wid], xd_v)
    pltpu.sync_copy(w_hbm, w_v)

    for c in copies:
        c.wait()

    full = pl.ds(0, L)
    for j in range(NCH):
        sl = pl.ds(j * L, L)
        a = xd_v[0, sl] * w_v[0, full]
        for d in range(1, NDF):
            a = a + xd_v[d, sl] * w_v[d, full]
        for f in range(NSF):
            a = a + emb_v[f, sl]
        acc_v[sl] = a

    pltpu.sync_copy(acc_v, out_hbm.at[pl.ds(base, TB)])


@jax.jit
def _run(X, table, W_dense):
    xs = X[:, :NSF].T.reshape(NSF, NW, TB).transpose(1, 0, 2)
    xd = X[:, NSF:].T.reshape(NDF, NW, TB).transpose(1, 0, 2)
    wb = jnp.broadcast_to(W_dense, (NDF, L))
    out = _linear_part(xs, xd, wb, table.reshape(-1))
    return out.reshape(B, 1)


def kernel(X, table, W_dense, sparse_col_idx, dense_col_idx):
    return _run(X, table, W_dense)
